# baseline (speedup 1.0000x reference)
import functools

import jax
import jax.numpy as jnp
from jax import lax
from jax.experimental import pallas as pl
from jax.experimental.pallas import tpu as pltpu
from jax.experimental.pallas import tpu_sc as plsc

NN = 100000
EE = 100000
DD = 128
NT = 32
PT = 3136
NP = NT * PT
CH = 112
NCH = PT // CH
CW_CH = 2048
EPAD = NP - EE
BN_EPS = 1e-5
EPS = 1e-5

_f32 = jnp.float32
_mesh = plsc.VectorSubcoreMesh(core_axis_name="c", subcore_axis_name="s")
_sc_params = pltpu.CompilerParams(needs_layout_passes=False)


def _wid():
    return lax.axis_index("s") * 2 + lax.axis_index("c")


def _k1_body(h_ref, e_ref, wr_ref, br_ref, wc_ref, bc_ref, wv_ref, vb_ref,
             xr_ref, xc_ref, hv_ref, ep_ref):
    i = pl.program_id(0)
    rows = i * 512 + lax.broadcasted_iota(jnp.int32, (512, 1), 0)
    m = rows < NN
    h_raw = h_ref[...]
    e_raw = e_ref[...]
    hb = jnp.where(m, h_raw, 0.0)
    eb = jnp.where(m, e_raw, 0.0)
    he = jnp.concatenate([hb, eb], axis=1)
    xr_ref[...] = jnp.dot(he, wr_ref[...], preferred_element_type=_f32) + br_ref[...]
    xc_ref[...] = jnp.where(
        m, jnp.dot(he, wc_ref[...], preferred_element_type=_f32) + bc_ref[...], -1e30)
    hv_ref[...] = jnp.dot(hb, wv_ref[...], preferred_element_type=_f32) + vb_ref[...]
    ep_ref[...] = jnp.where(m, e_raw, -1e30)


def _k1(h, e, wr, br, wc, bc, wv, vb):
    blk = lambda s: pl.BlockSpec(s, lambda i: (0,) * len(s))
    return pl.pallas_call(
        _k1_body,
        grid=(NP // 512,),
        in_specs=[
            pl.BlockSpec((512, DD), lambda i: (i, 0)),
            pl.BlockSpec((512, DD), lambda i: (i, 0)),
            blk((2 * DD, DD)), blk((1, DD)),
            blk((2 * DD, DD)), blk((1, DD)),
            blk((DD, DD)), blk((1, DD)),
        ],
        out_specs=[pl.BlockSpec((512, DD), lambda i: (i, 0))] * 4,
        out_shape=[jax.ShapeDtypeStruct((NP, DD), _f32)] * 4,
    )(h, e, wr, br, wc, bc, wv, vb)


@functools.partial(
    pl.kernel,
    out_type=jax.ShapeDtypeStruct((NP,), jnp.int32),
    mesh=_mesh,
    compiler_params=_sc_params,
    scratch_types=[
        pltpu.VMEM((PT,), jnp.int32),
        pltpu.VMEM((CW_CH,), jnp.int32),
        pltpu.VMEM((CW_CH,), jnp.int32),
    ],
)
def _kw(rowp_hbm, colp_hbm, cw_hbm, slab, rbuf, cbuf):
    wid = _wid()
    base = wid * PT
    neg1 = jnp.full((16,), -1, jnp.int32)
    iota = lax.iota(jnp.int32, 16)

    def init(i, _):
        slab[pl.ds(i * 16, 16)] = neg1
        return 0
    lax.fori_loop(0, PT // 16, init, 0)

    def chunk(cix, _):
        pltpu.sync_copy(rowp_hbm.at[pl.ds(cix * CW_CH, CW_CH)], rbuf)
        pltpu.sync_copy(colp_hbm.at[pl.ds(cix * CW_CH, CW_CH)], cbuf)
        kbase = cix * CW_CH

        def vreg(i, _):
            rv = rbuf[pl.ds(i * 16, 16)]
            cv = cbuf[pl.ds(i * 16, 16)]
            kvec = kbase + i * 16 + iota
            m = (rv >= base) & (rv < base + PT) & (kvec < EE)
            idx = jnp.clip(rv - base, 0, PT - 1)
            plsc.store_scatter(slab, [idx], cv, mask=m)
            return 0
        lax.fori_loop(0, CW_CH // 16, vreg, 0)
        return 0
    lax.fori_loop(0, NP // CW_CH, chunk, 0)
    pltpu.sync_copy(slab, cw_hbm.at[pl.ds(base, PT)])


@functools.partial(
    pl.kernel,
    out_type=jax.ShapeDtypeStruct((NT * 2 * DD,), _f32),
    mesh=_mesh,
    compiler_params=_sc_params,
    scratch_types=[
        pltpu.VMEM((CH,), jnp.int32),
        pltpu.VMEM((CH,), jnp.int32),
        pltpu.VMEM((CH, DD), _f32),
        pltpu.VMEM((CH, DD), _f32),
        pltpu.VMEM((2 * DD,), _f32),
        pltpu.SemaphoreType.DMA,
        pltpu.SemaphoreType.DMA,
    ],
)
def _k2(rowp_hbm, colp_hbm, xr_hbm, xc_hbm, st_hbm,
        ribuf, cibuf, xrb, xcb, outb, sem1, sem2):
    wid = _wid()
    base = wid * PT
    zero = jnp.zeros((16,), _f32)

    def chunk(cix, carry):
        off = base + cix * CH
        pltpu.sync_copy(rowp_hbm.at[pl.ds(off, CH)], ribuf)
        pltpu.sync_copy(colp_hbm.at[pl.ds(off, CH)], cibuf)
        cp1 = pltpu.async_copy(xr_hbm.at[ribuf], xrb, sem1)
        cp2 = pltpu.async_copy(xc_hbm.at[cibuf], xcb, sem2)
        cp1.wait()
        cp2.wait()

        def edge(i, car):
            s = list(car[0])
            q = list(car[1])
            for j in range(8):
                sl = pl.ds(j * 16, 16)
                t = xrb[i, sl] + xcb[i, sl]
                s[j] = s[j] + t
                q[j] = q[j] + t * t
            return (tuple(s), tuple(q))
        return lax.fori_loop(0, CH, edge, carry)

    init = (tuple(zero for _ in range(8)), tuple(zero for _ in range(8)))
    s, q = lax.fori_loop(0, NCH, chunk, init)
    for j in range(8):
        outb[pl.ds(j * 16, 16)] = s[j]
        outb[pl.ds(DD + j * 16, 16)] = q[j]
    pltpu.sync_copy(outb, st_hbm.at[pl.ds(wid * 2 * DD, 2 * DD)])


@functools.partial(
    pl.kernel,
    out_type=(jax.ShapeDtypeStruct((NP, DD), _f32),
              jax.ShapeDtypeStruct((NT * 2 * DD,), _f32)),
    mesh=_mesh,
    compiler_params=_sc_params,
    scratch_types=[
        pltpu.VMEM((NT * 2 * DD,), _f32),
        pltpu.VMEM((DD,), _f32),
        pltpu.VMEM((DD,), _f32),
        pltpu.VMEM((8, DD), _f32),
        pltpu.VMEM((8, DD), _f32),
        pltpu.VMEM((CH,), jnp.int32),
        pltpu.VMEM((CH,), jnp.int32),
        pltpu.VMEM((CH,), jnp.int32),
        pltpu.VMEM((CH, DD), _f32),
        pltpu.VMEM((CH, DD), _f32),
        pltpu.VMEM((CH, DD), _f32),
        pltpu.VMEM((CH, DD), _f32),
        pltpu.VMEM((CH, DD), _f32),
        pltpu.VMEM((2 * DD,), _f32),
        pltpu.SemaphoreType.DMA,
        pltpu.SemaphoreType.DMA,
    ],
)
def _k3(cw_hbm, colp_hbm, xr_hbm, xc_hbm, hv_hbm, ep_hbm, st_hbm, g_hbm, b_hbm,
        sig_hbm, cn_hbm,
        stbuf, gbuf, bbuf, x0buf, c0buf, cwb, idxb, colb,
        xrb, xcb, gvb, eb, sigb, outb, sem1, sem2):
    wid = _wid()
    base = wid * PT
    zero = jnp.zeros((16,), _f32)

    pltpu.sync_copy(st_hbm, stbuf)
    pltpu.sync_copy(g_hbm, gbuf)
    pltpu.sync_copy(b_hbm, bbuf)
    pltpu.sync_copy(xr_hbm.at[pl.ds(0, 8)], x0buf)
    pltpu.sync_copy(xc_hbm.at[pl.ds(0, 8)], c0buf)

    s_l, c_l = [], []
    for j in range(8):
        sl = pl.ds(j * 16, 16)

        def red(t2, car):
            return (car[0] + stbuf[pl.ds(t2 * 2 * DD + j * 16, 16)],
                    car[1] + stbuf[pl.ds(t2 * 2 * DD + DD + j * 16, 16)])
        ssum, qsum = lax.fori_loop(0, NT, red, (zero, zero))
        t0 = x0buf[0, sl] + c0buf[0, sl]
        ssum = ssum - float(EPAD) * t0
        qsum = qsum - float(EPAD) * (t0 * t0)
        mean = ssum * (1.0 / EE)
        var = qsum * (1.0 / EE) - mean * mean
        x = jnp.maximum(var, 0.0) + BN_EPS
        yi = 0x5F3759DF - lax.shift_right_logical(plsc.bitcast(x, jnp.int32), 1)
        y = plsc.bitcast(yi, _f32)
        for _ in range(4):
            y = y * (1.5 - 0.5 * x * y * y)
        sj = gbuf[sl] * y
        s_l.append(sj)
        c_l.append(bbuf[sl] - mean * sj)

    def chunk(cix, carry):
        noff = base + cix * CH
        pltpu.sync_copy(cw_hbm.at[pl.ds(noff, CH)], cwb)
        pltpu.sync_copy(colp_hbm.at[pl.ds(noff, CH)], colb)
        pltpu.sync_copy(ep_hbm.at[pl.ds(noff, CH)], eb)
        pltpu.sync_copy(xr_hbm.at[pl.ds(noff, CH)], xrb)

        def mk(i, _):
            cwv = cwb[pl.ds(i * 16, 16)]
            idxb[pl.ds(i * 16, 16)] = jnp.where(cwv < 0, NN, cwv)
            return 0
        lax.fori_loop(0, CH // 16, mk, 0)
        cp1 = pltpu.async_copy(xc_hbm.at[idxb], xcb, sem1)
        cp2 = pltpu.async_copy(hv_hbm.at[colb], gvb, sem2)
        cp1.wait()
        cp2.wait()

        def node(i, car):
            cs = list(car[0])
            nm = list(car[1])
            for j in range(8):
                sl = pl.ds(j * 16, 16)
                t = xrb[i, sl] + xcb[i, sl]
                bn = t * s_l[j] + c_l[j]
                add = jnp.maximum(bn, 0.0)
                pre = eb[i, sl] + add
                z = jnp.exp(-jnp.abs(pre))
                inv = 1.0 / (1.0 + z)
                sg = jnp.where(pre >= 0, inv, z * inv)
                sigb[i, sl] = sg
                cs[j] = cs[j] + sg
                nm[j] = nm[j] + sg * gvb[i, sl]
            return (tuple(cs), tuple(nm))
        carry = lax.fori_loop(0, CH, node, carry)
        pltpu.sync_copy(sigb, sig_hbm.at[pl.ds(noff, CH)])
        return carry

    init = (tuple(zero for _ in range(8)), tuple(zero for _ in range(8)))
    cs, nm = lax.fori_loop(0, NCH, chunk, init)
    for j in range(8):
        outb[pl.ds(j * 16, 16)] = cs[j]
        outb[pl.ds(DD + j * 16, 16)] = nm[j]
    pltpu.sync_copy(outb, cn_hbm.at[pl.ds(wid * 2 * DD, 2 * DD)])


def _k4_body(h_ref, sig_ref, cn_ref, wu_ref, ub_ref, hout_ref, enew_ref):
    cn = cn_ref[...]
    colsum = jnp.sum(cn[:, 0, :], axis=0)
    num = jnp.sum(cn[:, 1, :], axis=0)
    r = 1.0 / (colsum + EPS)
    enew_ref[...] = sig_ref[...] * r[None, :]
    hu = jnp.dot(h_ref[...], wu_ref[...], preferred_element_type=_f32) + ub_ref[...]
    hout_ref[...] = jnp.maximum(hu + (num * r)[None, :], 0.0)


def _k4(h, sig, cn, wu, ub):
    return pl.pallas_call(
        _k4_body,
        grid=(NN // 1000,),
        in_specs=[
            pl.BlockSpec((1000, DD), lambda i: (i, 0)),
            pl.BlockSpec((1000, DD), lambda i: (i, 0)),
            pl.BlockSpec((NT, 2, DD), lambda i: (0, 0, 0)),
            pl.BlockSpec((DD, DD), lambda i: (0, 0)),
            pl.BlockSpec((1, DD), lambda i: (0, 0)),
        ],
        out_specs=[pl.BlockSpec((1000, DD), lambda i: (i, 0))] * 2,
        out_shape=[jax.ShapeDtypeStruct((NN, DD), _f32)] * 2,
    )(h, sig, cn, wu, ub)


def kernel(h, e, edge_index, A_w, A_b, B_w, B_b, C_w, C_b, Dm_w, Dm_b,
           U_w, U_b, V_w, V_b, bn_g, bn_b):
    row = edge_index[0]
    col = edge_index[1]
    pad = jnp.zeros((NP - EE,), jnp.int32)
    rowp = jnp.concatenate([row, pad])
    colp = jnp.concatenate([col, pad])

    wr = jnp.concatenate([A_w.T, Dm_w.T], axis=0)
    wc = jnp.concatenate([B_w.T, C_w.T], axis=0)
    br = (A_b + Dm_b).reshape(1, DD)
    bc = (B_b + C_b).reshape(1, DD)
    vb = V_b.reshape(1, DD)
    ub = U_b.reshape(1, DD)

    cw = _kw(rowp, colp)
    xr, xc, hv, ep = _k1(h, e, wr, br, wc, bc, V_w.T, vb)
    st = _k2(rowp, colp, xr, xc)
    sig, cn = _k3(cw, colp, xr, xc, hv, ep, st, bn_g, bn_b)
    h_out, e_new = _k4(h, sig, cn.reshape(NT, 2, DD), U_w.T, ub)
    return (h_out, e_new)
